# Initial kernel scaffold; baseline (speedup 1.0000x reference)
import functools

import jax
import jax.numpy as jnp
from jax import lax
from jax.experimental import pallas as pl
from jax.experimental.pallas import tpu as pltpu
from jax.experimental.pallas import tpu_sc as plsc

N = 10000
E = 320000
F = 128
NC = 2
NS = 16
NW = NC * NS
EPW = E // NW
BK = 80
NB = EPW // BK
RPT = N // NS
ZR = 125
DW = 16


def _edge_pass_body(with_deg, *refs):
    if with_deg:
        (src_hbm, dst_hbm, table_hbm, pacc_hbm, pdeg_hbm,
         acc_s, deg_s, srcb, dstb, rows, onesb, zb, dz, sem) = refs
    else:
        (src_hbm, dst_hbm, table_hbm, pacc_hbm,
         acc_s, srcb, dstb, rows, zb, sem) = refs

    c = lax.axis_index("c")
    s = lax.axis_index("s")
    wid = s * NC + c

    z16 = jnp.zeros((16,), jnp.float32)

    def zrow(i, carry):
        for j in range(F // 16):
            zb[i, pl.ds(j * 16, 16)] = z16
        return carry
    lax.fori_loop(0, ZR, zrow, 0)

    for j in range(RPT // ZR):
        pltpu.sync_copy(zb, acc_s.at[pl.ds(s * RPT + j * ZR, ZR)])

    if with_deg:
        o16 = jnp.ones((16,), jnp.float32)

        def dzrow(i, carry):
            dz[i] = z16
            return carry
        lax.fori_loop(0, RPT, dzrow, 0)

        def orow(i, carry):
            onesb[i] = o16
            return carry
        lax.fori_loop(0, BK, orow, 0)
        pltpu.sync_copy(dz, deg_s.at[pl.ds(s * RPT, RPT)])

    plsc.subcore_barrier()

    ebase = wid * EPW

    def body(j, carry):
        off = pl.multiple_of(ebase + j * BK, 8)
        pltpu.sync_copy(src_hbm.at[pl.ds(off, BK)], srcb)
        pltpu.sync_copy(dst_hbm.at[pl.ds(off, BK)], dstb)
        pltpu.async_copy(table_hbm.at[srcb], rows, sem).wait()
        pltpu.sync_copy(rows, acc_s.at[dstb], add=True)
        if with_deg:
            pltpu.sync_copy(onesb, deg_s.at[dstb], add=True)
        return carry
    lax.fori_loop(0, NB, body, 0)

    plsc.subcore_barrier()

    pltpu.sync_copy(acc_s.at[pl.ds(s * RPT, RPT)],
                    pacc_hbm.at[c, pl.ds(s * RPT, RPT)])
    if with_deg:
        pltpu.sync_copy(deg_s.at[pl.ds(s * RPT, RPT)],
                        pdeg_hbm.at[c, pl.ds(s * RPT, RPT)])


_SC_MESH = plsc.VectorSubcoreMesh(
    core_axis_name="c", subcore_axis_name="s", num_cores=NC, num_subcores=NS)

_edge_pass_deg = functools.partial(
    pl.kernel, functools.partial(_edge_pass_body, True),
    out_type=[jax.ShapeDtypeStruct((NC, N, F), jnp.float32),
              jax.ShapeDtypeStruct((NC, N, DW), jnp.float32)],
    mesh=_SC_MESH,
    scratch_types=[
        pltpu.VMEM_SHARED((N, F), jnp.float32),
        pltpu.VMEM_SHARED((N, DW), jnp.float32),
        pltpu.VMEM((BK,), jnp.int32),
        pltpu.VMEM((BK,), jnp.int32),
        pltpu.VMEM((BK, F), jnp.float32),
        pltpu.VMEM((BK, DW), jnp.float32),
        pltpu.VMEM((ZR, F), jnp.float32),
        pltpu.VMEM((RPT, DW), jnp.float32),
        pltpu.SemaphoreType.DMA,
    ],
)()

_edge_pass = functools.partial(
    pl.kernel, functools.partial(_edge_pass_body, False),
    out_type=jax.ShapeDtypeStruct((NC, N, F), jnp.float32),
    mesh=_SC_MESH,
    scratch_types=[
        pltpu.VMEM_SHARED((N, F), jnp.float32),
        pltpu.VMEM((BK,), jnp.int32),
        pltpu.VMEM((BK,), jnp.int32),
        pltpu.VMEM((BK, F), jnp.float32),
        pltpu.VMEM((ZR, F), jnp.float32),
        pltpu.SemaphoreType.DMA,
    ],
)()

BR = 400


def _combine1_body(p_ref, d_ref, agg_ref):
    d = d_ref[0] + d_ref[1]
    deg = jnp.maximum(d[:, 0:1], 1.0)
    agg_ref[...] = (p_ref[0] + p_ref[1]) / deg


def _combine2_body(feat_ref, agg1_ref, p_ref, d_ref, out_ref):
    d = d_ref[0] + d_ref[1]
    deg = jnp.maximum(d[:, 0:1], 1.0)
    out_ref[:, 0:F] = feat_ref[...]
    out_ref[:, F:2 * F] = agg1_ref[...]
    out_ref[:, 2 * F:3 * F] = agg1_ref[...]
    out_ref[:, 3 * F:4 * F] = (p_ref[0] + p_ref[1]) / deg


_combine1 = pl.pallas_call(
    _combine1_body,
    grid=(N // BR,),
    in_specs=[
        pl.BlockSpec((NC, BR, F), lambda i: (0, i, 0)),
        pl.BlockSpec((NC, BR, DW), lambda i: (0, i, 0)),
    ],
    out_specs=pl.BlockSpec((BR, F), lambda i: (i, 0)),
    out_shape=jax.ShapeDtypeStruct((N, F), jnp.float32),
)

_combine2 = pl.pallas_call(
    _combine2_body,
    grid=(N // BR,),
    in_specs=[
        pl.BlockSpec((BR, F), lambda i: (i, 0)),
        pl.BlockSpec((BR, F), lambda i: (i, 0)),
        pl.BlockSpec((NC, BR, F), lambda i: (0, i, 0)),
        pl.BlockSpec((NC, BR, DW), lambda i: (0, i, 0)),
    ],
    out_specs=pl.BlockSpec((BR, 4 * F), lambda i: (i, 0)),
    out_shape=jax.ShapeDtypeStruct((N, 4 * F), jnp.float32),
)


def kernel(nodes, edge_index, features):
    src = edge_index[0]
    dst = edge_index[1]
    pacc1, pdeg = _edge_pass_deg(src, dst, features)
    agg1 = _combine1(pacc1, pdeg)
    pacc2 = _edge_pass(src, dst, agg1)
    return _combine2(features, agg1, pacc2, pdeg)


# speedup vs baseline: 3.0323x; 3.0323x over previous
import functools

import jax
import jax.numpy as jnp
from jax import lax
from jax.experimental import pallas as pl
from jax.experimental.pallas import tpu as pltpu
from jax.experimental.pallas import tpu_sc as plsc

N = 10000
E = 320000
F = 128
NC = 2
NS = 16
HALF = N // NC
AR = 5120
EPT = E // NS
BK = 80
NB = EPT // BK
ZR = 160
DW = 16


def _edge_pass_body(with_deg, *refs):
    if with_deg:
        (src_hbm, dst_hbm, table_hbm, acc_hbm, deg_hbm,
         acc_s, deg_s, srcb, dstb, dstb2, rows, onesb, zb, sem) = refs
    else:
        (src_hbm, dst_hbm, table_hbm, acc_hbm,
         acc_s, srcb, dstb, dstb2, rows, zb, sem) = refs

    c = lax.axis_index("c")
    s = lax.axis_index("s")

    z16 = jnp.zeros((16,), jnp.float32)

    def zrow(i, carry):
        for j in range(F // 16):
            zb[i, pl.ds(j * 16, 16)] = z16
        return carry
    lax.fori_loop(0, ZR, zrow, 0)

    zoff = pl.multiple_of(s * (2 * ZR), 8)
    pltpu.sync_copy(zb, acc_s.at[pl.ds(zoff, ZR)])
    pltpu.sync_copy(zb, acc_s.at[pl.ds(zoff + ZR, ZR)])

    if with_deg:
        o16 = jnp.ones((16,), jnp.float32)

        def orow(i, carry):
            for j in range(F // 16):
                onesb[i, pl.ds(j * 16, 16)] = o16
            return carry
        lax.fori_loop(0, BK, orow, 0)
        pltpu.sync_copy(zb, deg_s.at[pl.ds(zoff, ZR)])
        pltpu.sync_copy(zb, deg_s.at[pl.ds(zoff + ZR, ZR)])

    plsc.subcore_barrier()

    lo = c * HALF
    dump = HALF + s * 7
    ebase = s * EPT

    def body(j, carry):
        off = pl.multiple_of(ebase + j * BK, 8)
        pltpu.sync_copy(src_hbm.at[pl.ds(off, BK)], srcb)
        pltpu.sync_copy(dst_hbm.at[pl.ds(off, BK)], dstb)
        for k in range(BK // 16):
            d = dstb[pl.ds(k * 16, 16)]
            local = d - lo
            inb = (local >= 0) & (local < HALF)
            dstb2[pl.ds(k * 16, 16)] = jnp.where(inb, local, dump)
        pltpu.async_copy(table_hbm.at[srcb], rows, sem).wait()
        pltpu.sync_copy(rows, acc_s.at[dstb2], add=True)
        if with_deg:
            pltpu.sync_copy(onesb, deg_s.at[dstb2], add=True)
        return carry
    lax.fori_loop(0, NB, body, 0)

    plsc.subcore_barrier()

    @pl.when(s < HALF // 1000)
    def _writeback():
        soff = pl.multiple_of(s * 1000, 8)
        doff = pl.multiple_of(c * HALF + s * 1000, 8)
        pltpu.sync_copy(acc_s.at[pl.ds(soff, 1000)], acc_hbm.at[pl.ds(doff, 1000)])
        if with_deg:
            pltpu.sync_copy(deg_s.at[pl.ds(soff, 1000)],
                            deg_hbm.at[pl.ds(doff, 1000)])


_SC_MESH = plsc.VectorSubcoreMesh(
    core_axis_name="c", subcore_axis_name="s", num_cores=NC, num_subcores=NS)

_edge_pass_deg = functools.partial(
    pl.kernel, functools.partial(_edge_pass_body, True),
    out_type=[jax.ShapeDtypeStruct((N, F), jnp.float32),
              jax.ShapeDtypeStruct((N, F), jnp.float32)],
    mesh=_SC_MESH,
    scratch_types=[
        pltpu.VMEM_SHARED((AR, F), jnp.float32),
        pltpu.VMEM_SHARED((AR, F), jnp.float32),
        pltpu.VMEM((BK,), jnp.int32),
        pltpu.VMEM((BK,), jnp.int32),
        pltpu.VMEM((BK,), jnp.int32),
        pltpu.VMEM((BK, F), jnp.float32),
        pltpu.VMEM((BK, F), jnp.float32),
        pltpu.VMEM((ZR, F), jnp.float32),
        pltpu.SemaphoreType.DMA,
    ],
)()

_edge_pass = functools.partial(
    pl.kernel, functools.partial(_edge_pass_body, False),
    out_type=jax.ShapeDtypeStruct((N, F), jnp.float32),
    mesh=_SC_MESH,
    scratch_types=[
        pltpu.VMEM_SHARED((AR, F), jnp.float32),
        pltpu.VMEM((BK,), jnp.int32),
        pltpu.VMEM((BK,), jnp.int32),
        pltpu.VMEM((BK,), jnp.int32),
        pltpu.VMEM((BK, F), jnp.float32),
        pltpu.VMEM((ZR, F), jnp.float32),
        pltpu.SemaphoreType.DMA,
    ],
)()

BR = 400


def _combine1_body(p_ref, d_ref, agg_ref):
    deg = jnp.maximum(d_ref[:, 0:1], 1.0)
    agg_ref[...] = p_ref[...] / deg


def _combine2_body(feat_ref, agg1_ref, p_ref, d_ref, out_ref):
    deg = jnp.maximum(d_ref[:, 0:1], 1.0)
    out_ref[:, 0:F] = feat_ref[...]
    out_ref[:, F:2 * F] = agg1_ref[...]
    out_ref[:, 2 * F:3 * F] = agg1_ref[...]
    out_ref[:, 3 * F:4 * F] = p_ref[...] / deg


_combine1 = pl.pallas_call(
    _combine1_body,
    grid=(N // BR,),
    in_specs=[
        pl.BlockSpec((BR, F), lambda i: (i, 0)),
        pl.BlockSpec((BR, F), lambda i: (i, 0)),
    ],
    out_specs=pl.BlockSpec((BR, F), lambda i: (i, 0)),
    out_shape=jax.ShapeDtypeStruct((N, F), jnp.float32),
)

_combine2 = pl.pallas_call(
    _combine2_body,
    grid=(N // BR,),
    in_specs=[
        pl.BlockSpec((BR, F), lambda i: (i, 0)),
        pl.BlockSpec((BR, F), lambda i: (i, 0)),
        pl.BlockSpec((BR, F), lambda i: (i, 0)),
        pl.BlockSpec((BR, F), lambda i: (i, 0)),
    ],
    out_specs=pl.BlockSpec((BR, 4 * F), lambda i: (i, 0)),
    out_shape=jax.ShapeDtypeStruct((N, 4 * F), jnp.float32),
)


def kernel(nodes, edge_index, features):
    src = edge_index[0]
    dst = edge_index[1]
    acc1, deg = _edge_pass_deg(src, dst, features)
    agg1 = _combine1(acc1, deg)
    acc2 = _edge_pass(src, dst, agg1)
    return _combine2(features, agg1, acc2, deg)


# speedup vs baseline: 5.6462x; 1.8620x over previous
import functools

import jax
import jax.numpy as jnp
from jax import lax
from jax.experimental import pallas as pl
from jax.experimental.pallas import tpu as pltpu
from jax.experimental.pallas import tpu_sc as plsc

N = 10000
E = 320000
F = 128
NC = 2
NS = 16
HALF = N // NC
AR = 5040
EPT = E // NS
BK = 80
NB = EPT // BK


def _edge_pass_body(with_deg, *refs):
    if with_deg:
        (src_hbm, dst_hbm, table_hbm, agg_hbm, degb_hbm,
         acc_s, deg_s, srcb0, srcb1, dstb0, dstb1, d2a, d2b,
         rows0, rows1, onesb, isem0, isem1, gsem0, gsem1) = refs
    else:
        (src_hbm, dst_hbm, table_hbm, degb_hbm, agg_hbm,
         acc_s, srcb0, srcb1, dstb0, dstb1, d2a, d2b,
         rows0, rows1, isem0, isem1, gsem0, gsem1) = refs

    c = lax.axis_index("c")
    s = lax.axis_index("s")

    z16 = jnp.zeros((16,), jnp.float32)
    o16 = jnp.ones((16,), jnp.float32)

    def zrow(i, carry):
        for q in range(F // 16):
            rows0[i, pl.ds(q * 16, 16)] = z16
            rows1[i, pl.ds(q * 16, 16)] = z16
        return carry
    lax.fori_loop(0, BK, zrow, 0)

    zoff = s * (AR // NS)
    for j in range(3):
        pltpu.sync_copy(rows0, acc_s.at[pl.ds(zoff + j * BK, BK)])
    pltpu.sync_copy(rows0.at[pl.ds(0, 75)], acc_s.at[pl.ds(zoff + 240, 75)])

    if with_deg:
        for j in range(3):
            pltpu.sync_copy(rows1, deg_s.at[pl.ds(zoff + j * BK, BK)])
        pltpu.sync_copy(rows1.at[pl.ds(0, 75)],
                        deg_s.at[pl.ds(zoff + 240, 75)])

        def orow(i, carry):
            for q in range(F // 16):
                onesb[i, pl.ds(q * 16, 16)] = o16
            return carry
        lax.fori_loop(0, BK, orow, 0)

    plsc.subcore_barrier()

    lo = c * HALF
    dump = HALF + s * 2
    ebase = pl.multiple_of(s * EPT, 8)

    def idx_start(b, srcb, dstb, isem):
        off = pl.multiple_of(ebase + b * BK, 8)
        pltpu.async_copy(src_hbm.at[pl.ds(off, BK)], srcb, isem)
        pltpu.async_copy(dst_hbm.at[pl.ds(off, BK)], dstb, isem)

    def idx_wait(srcb, dstb, isem):
        pltpu.make_async_copy(src_hbm.at[pl.ds(0, BK)], srcb, isem).wait()
        pltpu.make_async_copy(dst_hbm.at[pl.ds(0, BK)], dstb, isem).wait()

    def compute_d2(dstb, d2ref):
        for k in range(BK // 16):
            d = dstb[pl.ds(k * 16, 16)]
            local = d - lo
            inb = (local >= 0) & (local < HALF)
            l2 = jnp.where(inb, local, dump)
            d2ref[pl.ds(k * 16, 16)] = l2

    def gather_start(srcb, rowsref, gsem):
        pltpu.async_copy(table_hbm.at[srcb], rowsref, gsem)

    def gather_wait(rowsref, gsem):
        pltpu.make_async_copy(table_hbm.at[srcb0], rowsref, gsem).wait()

    def scatter(rowsref, d2ref):
        pltpu.sync_copy(rowsref, acc_s.at[d2ref], add=True)
        if with_deg:
            pltpu.sync_copy(onesb, deg_s.at[d2ref], add=True)

    set0 = (srcb0, dstb0, d2a, rows0, isem0, gsem0)
    set1 = (srcb1, dstb1, d2b, rows1, isem1, gsem1)

    def half_iter(b_next, cur, nxt):
        csrc, cdst, cd2, crows, cisem, cgsem = cur
        nsrc, ndst, nd2, nrows, nisem, ngsem = nxt
        idx_wait(nsrc, ndst, nisem)
        compute_d2(ndst, nd2)
        gather_start(nsrc, nrows, ngsem)
        gather_wait(crows, cgsem)
        scatter(crows, cd2)
        idx_start(b_next, csrc, cdst, cisem)

    idx_start(0, srcb0, dstb0, isem0)
    idx_wait(srcb0, dstb0, isem0)
    compute_d2(dstb0, d2a)
    gather_start(srcb0, rows0, gsem0)
    idx_start(1, srcb1, dstb1, isem1)

    def body(k, carry):
        half_iter(2 * k + 2, set0, set1)
        half_iter(2 * k + 3, set1, set0)
        return carry
    lax.fori_loop(0, (NB - 2) // 2, body, 0)

    idx_wait(srcb1, dstb1, isem1)
    compute_d2(dstb1, d2b)
    gather_start(srcb1, rows1, gsem1)
    gather_wait(rows0, gsem0)
    scatter(rows0, d2a)
    gather_wait(rows1, gsem1)
    scatter(rows1, d2b)

    plsc.subcore_barrier()

    if with_deg:
        @pl.when(s < 8)
        def _divide():
            t640 = s * 640

            def grp(g, carry):
                base = t640 + g * 16

                def work(nrows):
                    pltpu.sync_copy(acc_s.at[pl.ds(base, 16)],
                                    rows1.at[pl.ds(0, 16)])
                    pltpu.sync_copy(deg_s.at[pl.ds(base, 16)],
                                    rows0.at[pl.ds(0, 16)])
                    for t in range(16):
                        b16 = 1.0 / jnp.maximum(rows0[t, pl.ds(0, 16)], 1.0)
                        for q in range(F // 16):
                            rows1[t, pl.ds(q * 16, 16)] = (
                                rows1[t, pl.ds(q * 16, 16)] * b16)
                            rows0[t, pl.ds(q * 16, 16)] = b16
                    doff = pl.multiple_of(c * HALF + base, 8)
                    pltpu.sync_copy(rows1.at[pl.ds(0, nrows)],
                                    agg_hbm.at[pl.ds(doff, nrows)])
                    pltpu.sync_copy(rows0.at[pl.ds(0, nrows)],
                                    degb_hbm.at[pl.ds(doff, nrows)])

                @pl.when(base + 16 <= HALF)
                def _full():
                    work(16)

                @pl.when((base < HALF) & (base + 16 > HALF))
                def _partial():
                    work(8)
                return carry
            lax.fori_loop(0, 40, grp, 0)
    else:
        @pl.when(s < 8)
        def _divide2():
            t640 = s * 640

            def blk(j, carry):
                base = t640 + j * 40

                @pl.when(base + 40 <= HALF)
                def _():
                    doff = pl.multiple_of(c * HALF + base, 8)
                    pltpu.sync_copy(acc_s.at[pl.ds(base, 40)],
                                    rows1.at[pl.ds(0, 40)])
                    pltpu.sync_copy(degb_hbm.at[pl.ds(doff, 40)],
                                    rows0.at[pl.ds(0, 40)])

                    def rowloop(i, carry2):
                        for q in range(F // 16):
                            rows1[i, pl.ds(q * 16, 16)] = (
                                rows1[i, pl.ds(q * 16, 16)]
                                * rows0[i, pl.ds(q * 16, 16)])
                        return carry2
                    lax.fori_loop(0, 40, rowloop, 0)
                    pltpu.sync_copy(rows1.at[pl.ds(0, 40)],
                                    agg_hbm.at[pl.ds(doff, 40)])
                return carry
            lax.fori_loop(0, 16, blk, 0)


_SC_MESH = plsc.VectorSubcoreMesh(
    core_axis_name="c", subcore_axis_name="s", num_cores=NC, num_subcores=NS)

_edge_pass_deg = functools.partial(
    pl.kernel, functools.partial(_edge_pass_body, True),
    out_type=[jax.ShapeDtypeStruct((N, F), jnp.float32),
              jax.ShapeDtypeStruct((N, F), jnp.float32)],
    mesh=_SC_MESH,
    scratch_types=[
        pltpu.VMEM_SHARED((AR, F), jnp.float32),
        pltpu.VMEM_SHARED((AR, F), jnp.float32),
        pltpu.VMEM((BK,), jnp.int32),
        pltpu.VMEM((BK,), jnp.int32),
        pltpu.VMEM((BK,), jnp.int32),
        pltpu.VMEM((BK,), jnp.int32),
        pltpu.VMEM((BK,), jnp.int32),
        pltpu.VMEM((BK,), jnp.int32),
        pltpu.VMEM((BK, F), jnp.float32),
        pltpu.VMEM((BK, F), jnp.float32),
        pltpu.VMEM((BK, F), jnp.float32),
        pltpu.SemaphoreType.DMA,
        pltpu.SemaphoreType.DMA,
        pltpu.SemaphoreType.DMA,
        pltpu.SemaphoreType.DMA,
    ],
)()

_edge_pass = functools.partial(
    pl.kernel, functools.partial(_edge_pass_body, False),
    out_type=jax.ShapeDtypeStruct((N, F), jnp.float32),
    mesh=_SC_MESH,
    scratch_types=[
        pltpu.VMEM_SHARED((AR, F), jnp.float32),
        pltpu.VMEM((BK,), jnp.int32),
        pltpu.VMEM((BK,), jnp.int32),
        pltpu.VMEM((BK,), jnp.int32),
        pltpu.VMEM((BK,), jnp.int32),
        pltpu.VMEM((BK,), jnp.int32),
        pltpu.VMEM((BK,), jnp.int32),
        pltpu.VMEM((BK, F), jnp.float32),
        pltpu.VMEM((BK, F), jnp.float32),
        pltpu.SemaphoreType.DMA,
        pltpu.SemaphoreType.DMA,
        pltpu.SemaphoreType.DMA,
        pltpu.SemaphoreType.DMA,
    ],
)()

BR = 400


def _assemble_body(feat_ref, agg1_ref, agg2_ref, out_ref):
    out_ref[:, 0:F] = feat_ref[...]
    out_ref[:, F:2 * F] = agg1_ref[...]
    out_ref[:, 2 * F:3 * F] = agg1_ref[...]
    out_ref[:, 3 * F:4 * F] = agg2_ref[...]


_assemble = pl.pallas_call(
    _assemble_body,
    grid=(N // BR,),
    in_specs=[
        pl.BlockSpec((BR, F), lambda i: (i, 0)),
        pl.BlockSpec((BR, F), lambda i: (i, 0)),
        pl.BlockSpec((BR, F), lambda i: (i, 0)),
    ],
    out_specs=pl.BlockSpec((BR, 4 * F), lambda i: (i, 0)),
    out_shape=jax.ShapeDtypeStruct((N, 4 * F), jnp.float32),
)


def kernel(nodes, edge_index, features):
    src = edge_index[0]
    dst = edge_index[1]
    agg1, degb = _edge_pass_deg(src, dst, features)
    agg2b = _edge_pass(src, dst, agg1, degb)
    return _assemble(features, agg1, agg2b)


# speedup vs baseline: 6.3724x; 1.1286x over previous
import functools

import jax
import jax.numpy as jnp
from jax import lax
from jax.experimental import pallas as pl
from jax.experimental.pallas import tpu as pltpu
from jax.experimental.pallas import tpu_sc as plsc

N = 10000
E = 320000
F = 128
NC = 2
NS = 16
HALF = N // NC
AR = 5040
EPT = E // NS
BK = 80
NB = EPT // BK


def _edge_pass_body(with_deg, *refs):
    if with_deg:
        (src_hbm, dst_hbm, table_hbm, agg_hbm, degb_hbm,
         acc_s, deg_s, srcb0, srcb1, dstb0, dstb1, d2a, d2b,
         rows0, rows1, onesb, isem0, isem1, gsem0, gsem1,
         ssem0, ssem1, dsem0, dsem1) = refs
    else:
        (src_hbm, dst_hbm, table_hbm, degb_hbm, agg_hbm,
         acc_s, srcb0, srcb1, dstb0, dstb1, d2a, d2b,
         rows0, rows1, isem0, isem1, gsem0, gsem1, ssem0, ssem1) = refs

    c = lax.axis_index("c")
    s = lax.axis_index("s")

    z16 = jnp.zeros((16,), jnp.float32)
    o16 = jnp.ones((16,), jnp.float32)

    def zrow(i, carry):
        for q in range(F // 16):
            rows0[i, pl.ds(q * 16, 16)] = z16
            rows1[i, pl.ds(q * 16, 16)] = z16
        return carry
    lax.fori_loop(0, BK, zrow, 0)

    zoff = s * (AR // NS)
    for j in range(3):
        pltpu.sync_copy(rows0, acc_s.at[pl.ds(zoff + j * BK, BK)])
    pltpu.sync_copy(rows0.at[pl.ds(0, 75)], acc_s.at[pl.ds(zoff + 240, 75)])

    if with_deg:
        for j in range(3):
            pltpu.sync_copy(rows1, deg_s.at[pl.ds(zoff + j * BK, BK)])
        pltpu.sync_copy(rows1.at[pl.ds(0, 75)],
                        deg_s.at[pl.ds(zoff + 240, 75)])

        def orow(i, carry):
            for q in range(F // 16):
                onesb[i, pl.ds(q * 16, 16)] = o16
            return carry
        lax.fori_loop(0, BK, orow, 0)

    plsc.subcore_barrier()

    lo = c * HALF
    dump = HALF + s * 2
    ebase = pl.multiple_of(s * EPT, 8)

    def idx_start(b, srcb, dstb, isem):
        off = pl.multiple_of(ebase + b * BK, 8)
        pltpu.async_copy(src_hbm.at[pl.ds(off, BK)], srcb, isem)
        pltpu.async_copy(dst_hbm.at[pl.ds(off, BK)], dstb, isem)

    def idx_wait(srcb, dstb, isem):
        pltpu.make_async_copy(src_hbm.at[pl.ds(0, BK)], srcb, isem).wait()
        pltpu.make_async_copy(dst_hbm.at[pl.ds(0, BK)], dstb, isem).wait()

    def compute_d2(dstb, d2ref):
        for k in range(BK // 16):
            d = dstb[pl.ds(k * 16, 16)]
            local = d - lo
            inb = (local >= 0) & (local < HALF)
            l2 = jnp.where(inb, local, dump)
            d2ref[pl.ds(k * 16, 16)] = l2

    def gather_start(srcb, rowsref, gsem):
        pltpu.async_copy(table_hbm.at[srcb], rowsref, gsem)

    def gather_wait(rowsref, gsem):
        pltpu.make_async_copy(table_hbm.at[srcb0], rowsref, gsem).wait()

    def scatter_start(rowsref, d2ref, ssem):
        pltpu.async_copy(rowsref, acc_s.at[d2ref], ssem, add=True)

    def scatter_wait(rowsref, d2ref, ssem):
        pltpu.make_async_copy(rowsref, acc_s.at[d2ref], ssem).wait()

    def deg_start(d2ref, dsem):
        pltpu.async_copy(onesb, deg_s.at[d2ref], dsem, add=True)

    def deg_wait(d2ref, dsem):
        pltpu.make_async_copy(onesb, deg_s.at[d2ref], dsem).wait()

    if with_deg:
        set0 = (srcb0, dstb0, d2a, rows0, isem0, gsem0, ssem0, dsem0)
        set1 = (srcb1, dstb1, d2b, rows1, isem1, gsem1, ssem1, dsem1)
    else:
        set0 = (srcb0, dstb0, d2a, rows0, isem0, gsem0, ssem0, None)
        set1 = (srcb1, dstb1, d2b, rows1, isem1, gsem1, ssem1, None)

    def half_iter(b_next, cur, nxt):
        csrc, cdst, cd2, crows, cisem, cgsem, cssem, cdsem = cur
        nsrc, ndst, nd2, nrows, nisem, ngsem, nssem, ndsem = nxt
        idx_wait(nsrc, ndst, nisem)
        if with_deg:
            deg_wait(nd2, ndsem)
        compute_d2(ndst, nd2)
        scatter_wait(nrows, nd2, nssem)
        gather_start(nsrc, nrows, ngsem)
        gather_wait(crows, cgsem)
        scatter_start(crows, cd2, cssem)
        if with_deg:
            deg_start(cd2, cdsem)
        idx_start(b_next, csrc, cdst, cisem)

    idx_start(0, srcb0, dstb0, isem0)
    dfill = jnp.full((16,), dump, jnp.int32)
    for k in range(BK // 16):
        d2b[pl.ds(k * 16, 16)] = dfill
    scatter_start(rows1, d2b, ssem1)
    if with_deg:
        deg_start(d2b, dsem1)
    idx_wait(srcb0, dstb0, isem0)
    compute_d2(dstb0, d2a)
    gather_start(srcb0, rows0, gsem0)
    idx_start(1, srcb1, dstb1, isem1)

    def body(k, carry):
        half_iter(2 * k + 2, set0, set1)
        half_iter(2 * k + 3, set1, set0)
        return carry
    lax.fori_loop(0, (NB - 2) // 2, body, 0)

    idx_wait(srcb1, dstb1, isem1)
    if with_deg:
        deg_wait(d2b, dsem1)
    compute_d2(dstb1, d2b)
    scatter_wait(rows1, d2b, ssem1)
    gather_start(srcb1, rows1, gsem1)
    gather_wait(rows0, gsem0)
    scatter_start(rows0, d2a, ssem0)
    if with_deg:
        deg_start(d2a, dsem0)
    gather_wait(rows1, gsem1)
    scatter_start(rows1, d2b, ssem1)
    if with_deg:
        deg_start(d2b, dsem1)

    scatter_wait(rows0, d2a, ssem0)
    scatter_wait(rows1, d2b, ssem1)
    if with_deg:
        deg_wait(d2a, dsem0)
        deg_wait(d2b, dsem1)

    plsc.subcore_barrier()

    if with_deg:
        @pl.when(s < 8)
        def _divide():
            t640 = s * 640

            def grp(g, carry):
                base = t640 + g * 16

                def work(nrows):
                    pltpu.sync_copy(acc_s.at[pl.ds(base, 16)],
                                    rows1.at[pl.ds(0, 16)])
                    pltpu.sync_copy(deg_s.at[pl.ds(base, 16)],
                                    rows0.at[pl.ds(0, 16)])
                    for t in range(16):
                        b16 = 1.0 / jnp.maximum(rows0[t, pl.ds(0, 16)], 1.0)
                        for q in range(F // 16):
                            rows1[t, pl.ds(q * 16, 16)] = (
                                rows1[t, pl.ds(q * 16, 16)] * b16)
                            rows0[t, pl.ds(q * 16, 16)] = b16
                    doff = pl.multiple_of(c * HALF + base, 8)
                    pltpu.sync_copy(rows1.at[pl.ds(0, nrows)],
                                    agg_hbm.at[pl.ds(doff, nrows)])
                    pltpu.sync_copy(rows0.at[pl.ds(0, nrows)],
                                    degb_hbm.at[pl.ds(doff, nrows)])

                @pl.when(base + 16 <= HALF)
                def _full():
                    work(16)

                @pl.when((base < HALF) & (base + 16 > HALF))
                def _partial():
                    work(8)
                return carry
            lax.fori_loop(0, 40, grp, 0)
    else:
        @pl.when(s < 8)
        def _divide2():
            t640 = s * 640

            def blk(j, carry):
                base = t640 + j * 40

                @pl.when(base + 40 <= HALF)
                def _():
                    doff = pl.multiple_of(c * HALF + base, 8)
                    pltpu.sync_copy(acc_s.at[pl.ds(base, 40)],
                                    rows1.at[pl.ds(0, 40)])
                    pltpu.sync_copy(degb_hbm.at[pl.ds(doff, 40)],
                                    rows0.at[pl.ds(0, 40)])

                    def rowloop(i, carry2):
                        for q in range(F // 16):
                            rows1[i, pl.ds(q * 16, 16)] = (
                                rows1[i, pl.ds(q * 16, 16)]
                                * rows0[i, pl.ds(q * 16, 16)])
                        return carry2
                    lax.fori_loop(0, 40, rowloop, 0)
                    pltpu.sync_copy(rows1.at[pl.ds(0, 40)],
                                    agg_hbm.at[pl.ds(doff, 40)])
                return carry
            lax.fori_loop(0, 16, blk, 0)


_SC_MESH = plsc.VectorSubcoreMesh(
    core_axis_name="c", subcore_axis_name="s", num_cores=NC, num_subcores=NS)

_edge_pass_deg = functools.partial(
    pl.kernel, functools.partial(_edge_pass_body, True),
    out_type=[jax.ShapeDtypeStruct((N, F), jnp.float32),
              jax.ShapeDtypeStruct((N, F), jnp.float32)],
    mesh=_SC_MESH,
    scratch_types=[
        pltpu.VMEM_SHARED((AR, F), jnp.float32),
        pltpu.VMEM_SHARED((AR, F), jnp.float32),
        pltpu.VMEM((BK,), jnp.int32),
        pltpu.VMEM((BK,), jnp.int32),
        pltpu.VMEM((BK,), jnp.int32),
        pltpu.VMEM((BK,), jnp.int32),
        pltpu.VMEM((BK,), jnp.int32),
        pltpu.VMEM((BK,), jnp.int32),
        pltpu.VMEM((BK, F), jnp.float32),
        pltpu.VMEM((BK, F), jnp.float32),
        pltpu.VMEM((BK, F), jnp.float32),
        pltpu.SemaphoreType.DMA,
        pltpu.SemaphoreType.DMA,
        pltpu.SemaphoreType.DMA,
        pltpu.SemaphoreType.DMA,
        pltpu.SemaphoreType.DMA,
        pltpu.SemaphoreType.DMA,
        pltpu.SemaphoreType.DMA,
        pltpu.SemaphoreType.DMA,
    ],
)()

_edge_pass = functools.partial(
    pl.kernel, functools.partial(_edge_pass_body, False),
    out_type=jax.ShapeDtypeStruct((N, F), jnp.float32),
    mesh=_SC_MESH,
    scratch_types=[
        pltpu.VMEM_SHARED((AR, F), jnp.float32),
        pltpu.VMEM((BK,), jnp.int32),
        pltpu.VMEM((BK,), jnp.int32),
        pltpu.VMEM((BK,), jnp.int32),
        pltpu.VMEM((BK,), jnp.int32),
        pltpu.VMEM((BK,), jnp.int32),
        pltpu.VMEM((BK,), jnp.int32),
        pltpu.VMEM((BK, F), jnp.float32),
        pltpu.VMEM((BK, F), jnp.float32),
        pltpu.SemaphoreType.DMA,
        pltpu.SemaphoreType.DMA,
        pltpu.SemaphoreType.DMA,
        pltpu.SemaphoreType.DMA,
        pltpu.SemaphoreType.DMA,
        pltpu.SemaphoreType.DMA,
    ],
)()

BR = 400


def _assemble_body(feat_ref, agg1_ref, agg2_ref, out_ref):
    out_ref[:, 0:F] = feat_ref[...]
    out_ref[:, F:2 * F] = agg1_ref[...]
    out_ref[:, 2 * F:3 * F] = agg1_ref[...]
    out_ref[:, 3 * F:4 * F] = agg2_ref[...]


_assemble = pl.pallas_call(
    _assemble_body,
    grid=(N // BR,),
    in_specs=[
        pl.BlockSpec((BR, F), lambda i: (i, 0)),
        pl.BlockSpec((BR, F), lambda i: (i, 0)),
        pl.BlockSpec((BR, F), lambda i: (i, 0)),
    ],
    out_specs=pl.BlockSpec((BR, 4 * F), lambda i: (i, 0)),
    out_shape=jax.ShapeDtypeStruct((N, 4 * F), jnp.float32),
)


def kernel(nodes, edge_index, features):
    src = edge_index[0]
    dst = edge_index[1]
    agg1, degb = _edge_pass_deg(src, dst, features)
    agg2b = _edge_pass(src, dst, agg1, degb)
    return _assemble(features, agg1, agg2b)


# speedup vs baseline: 6.6660x; 1.0461x over previous
import functools

import jax
import jax.numpy as jnp
from jax import lax
from jax.experimental import pallas as pl
from jax.experimental.pallas import tpu as pltpu
from jax.experimental.pallas import tpu_sc as plsc

N = 10000
E = 320000
F = 128
NC = 2
NS = 16
HALF = N // NC
AR = 5040
EPT = E // NS
BK = 80
NB = EPT // BK


def _edge_pass_body(with_deg, *refs):
    if with_deg:
        (src_hbm, dst_hbm, table_hbm, agg_hbm, degb_hbm,
         acc_s, deg_s, srcb0, srcb1, dstb0, dstb1, d2a, d2b,
         rows0, rows1, onesb, isem0, isem1, gsem0, gsem1,
         ssem0, ssem1, dsem0, dsem1) = refs
    else:
        (src_hbm, dst_hbm, table_hbm, degb_hbm, agg_hbm,
         acc_s, srcb0, srcb1, dstb0, dstb1, d2a, d2b,
         rows0, rows1, isem0, isem1, gsem0, gsem1, ssem0, ssem1) = refs

    c = lax.axis_index("c")
    s = lax.axis_index("s")

    z16 = jnp.zeros((16,), jnp.float32)
    o16 = jnp.ones((16,), jnp.float32)

    def zrow(i, carry):
        for q in range(F // 16):
            rows0[i, pl.ds(q * 16, 16)] = z16
            rows1[i, pl.ds(q * 16, 16)] = z16
        return carry
    lax.fori_loop(0, BK, zrow, 0)

    zoff = s * (AR // NS)
    for j in range(3):
        pltpu.sync_copy(rows0, acc_s.at[pl.ds(zoff + j * BK, BK)])
    pltpu.sync_copy(rows0.at[pl.ds(0, 75)], acc_s.at[pl.ds(zoff + 240, 75)])

    if with_deg:
        for j in range(3):
            pltpu.sync_copy(rows1, deg_s.at[pl.ds(zoff + j * BK, BK)])
        pltpu.sync_copy(rows1.at[pl.ds(0, 75)],
                        deg_s.at[pl.ds(zoff + 240, 75)])

        def orow(i, carry):
            for q in range(F // 16):
                onesb[i, pl.ds(q * 16, 16)] = o16
            return carry
        lax.fori_loop(0, BK, orow, 0)

    plsc.subcore_barrier()

    lo = c * HALF
    dump = HALF + s * 2
    ebase = pl.multiple_of(s * EPT, 8)

    def idx_start(b, srcb, dstb, isem):
        off = pl.multiple_of(ebase + b * BK, 8)
        pltpu.async_copy(src_hbm.at[pl.ds(off, BK)], srcb, isem)
        pltpu.async_copy(dst_hbm.at[pl.ds(off, BK)], dstb, isem)

    def idx_wait(srcb, dstb, isem):
        pltpu.make_async_copy(src_hbm.at[pl.ds(0, BK)], srcb, isem).wait()
        pltpu.make_async_copy(dst_hbm.at[pl.ds(0, BK)], dstb, isem).wait()

    def compute_d2(dstb, d2ref):
        for k in range(BK // 16):
            d = dstb[pl.ds(k * 16, 16)]
            local = d - lo
            inb = (local >= 0) & (local < HALF)
            l2 = jnp.where(inb, local, dump)
            d2ref[pl.ds(k * 16, 16)] = l2

    def gather_start(srcb, rowsref, gsem):
        pltpu.async_copy(table_hbm.at[srcb], rowsref, gsem)

    def gather_wait(rowsref, gsem):
        pltpu.make_async_copy(table_hbm.at[srcb0], rowsref, gsem).wait()

    def scatter_start(rowsref, d2ref, ssem):
        pltpu.async_copy(rowsref, acc_s.at[d2ref], ssem, add=True)

    def scatter_wait(rowsref, d2ref, ssem):
        pltpu.make_async_copy(rowsref, acc_s.at[d2ref], ssem).wait()

    def deg_start(d2ref, dsem):
        pltpu.async_copy(onesb, deg_s.at[d2ref], dsem, add=True)

    def deg_wait(d2ref, dsem):
        pltpu.make_async_copy(onesb, deg_s.at[d2ref], dsem).wait()

    if with_deg:
        set0 = (srcb0, dstb0, d2a, rows0, isem0, gsem0, ssem0, dsem0)
        set1 = (srcb1, dstb1, d2b, rows1, isem1, gsem1, ssem1, dsem1)
    else:
        set0 = (srcb0, dstb0, d2a, rows0, isem0, gsem0, ssem0, None)
        set1 = (srcb1, dstb1, d2b, rows1, isem1, gsem1, ssem1, None)

    def half_iter(b_next, cur, nxt):
        csrc, cdst, cd2, crows, cisem, cgsem, cssem, cdsem = cur
        nsrc, ndst, nd2, nrows, nisem, ngsem, nssem, ndsem = nxt
        idx_wait(nsrc, ndst, nisem)
        if with_deg:
            deg_wait(nd2, ndsem)
        compute_d2(ndst, nd2)
        scatter_wait(nrows, nd2, nssem)
        gather_start(nsrc, nrows, ngsem)
        gather_wait(crows, cgsem)
        scatter_start(crows, cd2, cssem)
        if with_deg:
            deg_start(cd2, cdsem)
        idx_start(b_next, csrc, cdst, cisem)

    idx_start(0, srcb0, dstb0, isem0)
    dfill = jnp.full((16,), dump, jnp.int32)
    for k in range(BK // 16):
        d2b[pl.ds(k * 16, 16)] = dfill
    scatter_start(rows1, d2b, ssem1)
    if with_deg:
        deg_start(d2b, dsem1)
    idx_wait(srcb0, dstb0, isem0)
    compute_d2(dstb0, d2a)
    gather_start(srcb0, rows0, gsem0)
    idx_start(1, srcb1, dstb1, isem1)

    def body(k, carry):
        half_iter(2 * k + 2, set0, set1)
        half_iter(2 * k + 3, set1, set0)
        return carry
    lax.fori_loop(0, (NB - 2) // 2, body, 0)

    idx_wait(srcb1, dstb1, isem1)
    if with_deg:
        deg_wait(d2b, dsem1)
    compute_d2(dstb1, d2b)
    scatter_wait(rows1, d2b, ssem1)
    gather_start(srcb1, rows1, gsem1)
    gather_wait(rows0, gsem0)
    scatter_start(rows0, d2a, ssem0)
    if with_deg:
        deg_start(d2a, dsem0)
    gather_wait(rows1, gsem1)
    scatter_start(rows1, d2b, ssem1)
    if with_deg:
        deg_start(d2b, dsem1)

    scatter_wait(rows0, d2a, ssem0)
    scatter_wait(rows1, d2b, ssem1)
    if with_deg:
        deg_wait(d2a, dsem0)
        deg_wait(d2b, dsem1)

    plsc.subcore_barrier()

    if with_deg:
        def _divide():
            t320 = s * 320

            def grp(g, carry):
                base = t320 + g * 16

                def work(nrows):
                    pltpu.sync_copy(acc_s.at[pl.ds(base, 16)],
                                    rows1.at[pl.ds(0, 16)])
                    pltpu.sync_copy(deg_s.at[pl.ds(base, 16)],
                                    rows0.at[pl.ds(0, 16)])
                    for t in range(16):
                        b16 = 1.0 / jnp.maximum(rows0[t, pl.ds(0, 16)], 1.0)
                        for q in range(F // 16):
                            rows1[t, pl.ds(q * 16, 16)] = (
                                rows1[t, pl.ds(q * 16, 16)] * b16)
                            rows0[t, pl.ds(q * 16, 16)] = b16
                    doff = pl.multiple_of(c * HALF + base, 8)
                    pltpu.sync_copy(rows1.at[pl.ds(0, nrows)],
                                    agg_hbm.at[pl.ds(doff, nrows)])
                    pltpu.sync_copy(rows0.at[pl.ds(0, nrows)],
                                    degb_hbm.at[pl.ds(doff, nrows)])

                @pl.when(base + 16 <= HALF)
                def _full():
                    work(16)

                @pl.when((base < HALF) & (base + 16 > HALF))
                def _partial():
                    work(8)
                return carry
            lax.fori_loop(0, 20, grp, 0)
        _divide()
    else:
        def _divide2():
            t320 = s * 320

            def blk(j, carry):
                base = t320 + j * 40

                @pl.when(base + 40 <= HALF)
                def _():
                    doff = pl.multiple_of(c * HALF + base, 8)
                    pltpu.sync_copy(acc_s.at[pl.ds(base, 40)],
                                    rows1.at[pl.ds(0, 40)])
                    pltpu.sync_copy(degb_hbm.at[pl.ds(doff, 40)],
                                    rows0.at[pl.ds(0, 40)])

                    def rowloop(i, carry2):
                        for q in range(F // 16):
                            rows1[i, pl.ds(q * 16, 16)] = (
                                rows1[i, pl.ds(q * 16, 16)]
                                * rows0[i, pl.ds(q * 16, 16)])
                        return carry2
                    lax.fori_loop(0, 40, rowloop, 0)
                    pltpu.sync_copy(rows1.at[pl.ds(0, 40)],
                                    agg_hbm.at[pl.ds(doff, 40)])
                return carry
            lax.fori_loop(0, 8, blk, 0)
        _divide2()


_SC_MESH = plsc.VectorSubcoreMesh(
    core_axis_name="c", subcore_axis_name="s", num_cores=NC, num_subcores=NS)

_edge_pass_deg = functools.partial(
    pl.kernel, functools.partial(_edge_pass_body, True),
    out_type=[jax.ShapeDtypeStruct((N, F), jnp.float32),
              jax.ShapeDtypeStruct((N, F), jnp.float32)],
    mesh=_SC_MESH,
    scratch_types=[
        pltpu.VMEM_SHARED((AR, F), jnp.float32),
        pltpu.VMEM_SHARED((AR, F), jnp.float32),
        pltpu.VMEM((BK,), jnp.int32),
        pltpu.VMEM((BK,), jnp.int32),
        pltpu.VMEM((BK,), jnp.int32),
        pltpu.VMEM((BK,), jnp.int32),
        pltpu.VMEM((BK,), jnp.int32),
        pltpu.VMEM((BK,), jnp.int32),
        pltpu.VMEM((BK, F), jnp.float32),
        pltpu.VMEM((BK, F), jnp.float32),
        pltpu.VMEM((BK, F), jnp.float32),
        pltpu.SemaphoreType.DMA,
        pltpu.SemaphoreType.DMA,
        pltpu.SemaphoreType.DMA,
        pltpu.SemaphoreType.DMA,
        pltpu.SemaphoreType.DMA,
        pltpu.SemaphoreType.DMA,
        pltpu.SemaphoreType.DMA,
        pltpu.SemaphoreType.DMA,
    ],
)()

_edge_pass = functools.partial(
    pl.kernel, functools.partial(_edge_pass_body, False),
    out_type=jax.ShapeDtypeStruct((N, F), jnp.float32),
    mesh=_SC_MESH,
    scratch_types=[
        pltpu.VMEM_SHARED((AR, F), jnp.float32),
        pltpu.VMEM((BK,), jnp.int32),
        pltpu.VMEM((BK,), jnp.int32),
        pltpu.VMEM((BK,), jnp.int32),
        pltpu.VMEM((BK,), jnp.int32),
        pltpu.VMEM((BK,), jnp.int32),
        pltpu.VMEM((BK,), jnp.int32),
        pltpu.VMEM((BK, F), jnp.float32),
        pltpu.VMEM((BK, F), jnp.float32),
        pltpu.SemaphoreType.DMA,
        pltpu.SemaphoreType.DMA,
        pltpu.SemaphoreType.DMA,
        pltpu.SemaphoreType.DMA,
        pltpu.SemaphoreType.DMA,
        pltpu.SemaphoreType.DMA,
    ],
)()

BR = 400


def _assemble_body(feat_ref, agg1_ref, agg2_ref, out_ref):
    out_ref[:, 0:F] = feat_ref[...]
    out_ref[:, F:2 * F] = agg1_ref[...]
    out_ref[:, 2 * F:3 * F] = agg1_ref[...]
    out_ref[:, 3 * F:4 * F] = agg2_ref[...]


_assemble = pl.pallas_call(
    _assemble_body,
    grid=(N // BR,),
    in_specs=[
        pl.BlockSpec((BR, F), lambda i: (i, 0)),
        pl.BlockSpec((BR, F), lambda i: (i, 0)),
        pl.BlockSpec((BR, F), lambda i: (i, 0)),
    ],
    out_specs=pl.BlockSpec((BR, 4 * F), lambda i: (i, 0)),
    out_shape=jax.ShapeDtypeStruct((N, 4 * F), jnp.float32),
)


def kernel(nodes, edge_index, features):
    src = edge_index[0]
    dst = edge_index[1]
    agg1, degb = _edge_pass_deg(src, dst, features)
    agg2b = _edge_pass(src, dst, agg1, degb)
    return _assemble(features, agg1, agg2b)


# speedup vs baseline: 6.7789x; 1.0169x over previous
import functools

import jax
import jax.numpy as jnp
from jax import lax
from jax.experimental import pallas as pl
from jax.experimental.pallas import tpu as pltpu
from jax.experimental.pallas import tpu_sc as plsc

N = 10000
E = 320000
F = 128
NC = 2
NS = 16
HALF = N // NC
AR = 5040
EPT = E // NS
BK = 80
NB = EPT // BK


def _edge_pass_body(with_deg, *refs):
    if with_deg:
        (src_hbm, dst_hbm, table_hbm, agg_hbm, degb_hbm,
         acc_s, deg_s, srcb0, srcb1, dstb0, dstb1, d2a, d2b,
         rows0, rows1, onesb, isem0, isem1, gsem0, gsem1,
         ssem0, ssem1, dsem0, dsem1) = refs
    else:
        (src_hbm, dst_hbm, table_hbm, degb_hbm, agg_hbm,
         acc_s, srcb0, srcb1, dstb0, dstb1, d2a, d2b,
         rows0, rows1, isem0, isem1, gsem0, gsem1, ssem0, ssem1) = refs

    c = lax.axis_index("c")
    s = lax.axis_index("s")

    z16 = jnp.zeros((16,), jnp.float32)
    o16 = jnp.ones((16,), jnp.float32)

    def zrow(i, carry):
        for q in range(F // 16):
            rows0[i, pl.ds(q * 16, 16)] = z16
            rows1[i, pl.ds(q * 16, 16)] = z16
        return carry
    lax.fori_loop(0, BK, zrow, 0)

    zoff = s * (AR // NS)
    for j in range(3):
        pltpu.sync_copy(rows0, acc_s.at[pl.ds(zoff + j * BK, BK)])
    pltpu.sync_copy(rows0.at[pl.ds(0, 75)], acc_s.at[pl.ds(zoff + 240, 75)])

    if with_deg:
        for j in range(3):
            pltpu.sync_copy(rows1, deg_s.at[pl.ds(zoff + j * BK, BK)])
        pltpu.sync_copy(rows1.at[pl.ds(0, 75)],
                        deg_s.at[pl.ds(zoff + 240, 75)])

        def orow(i, carry):
            for q in range(F // 16):
                onesb[i, pl.ds(q * 16, 16)] = o16
            return carry
        lax.fori_loop(0, BK, orow, 0)

    plsc.subcore_barrier()

    lo = c * HALF
    dump = HALF + s * 2
    ebase = pl.multiple_of(s * EPT, 8)

    def idx_start(b, srcb, dstb, isem):
        off = pl.multiple_of(ebase + b * BK, 8)
        pltpu.async_copy(src_hbm.at[pl.ds(off, BK)], srcb, isem)
        pltpu.async_copy(dst_hbm.at[pl.ds(off, BK)], dstb, isem)

    def idx_wait(srcb, dstb, isem):
        pltpu.make_async_copy(src_hbm.at[pl.ds(0, BK)], srcb, isem).wait()
        pltpu.make_async_copy(dst_hbm.at[pl.ds(0, BK)], dstb, isem).wait()

    def compute_d2(dstb, d2ref):
        for k in range(BK // 16):
            d = dstb[pl.ds(k * 16, 16)]
            local = d - lo
            inb = (local >= 0) & (local < HALF)
            l2 = jnp.where(inb, local, dump)
            d2ref[pl.ds(k * 16, 16)] = l2

    def gather_start(srcb, rowsref, gsem):
        pltpu.async_copy(table_hbm.at[srcb], rowsref, gsem)

    def gather_wait(rowsref, gsem):
        pltpu.make_async_copy(table_hbm.at[srcb0], rowsref, gsem).wait()

    def scatter_start(rowsref, d2ref, ssem):
        pltpu.async_copy(rowsref, acc_s.at[d2ref], ssem, add=True)

    def scatter_wait(rowsref, d2ref, ssem):
        pltpu.make_async_copy(rowsref, acc_s.at[d2ref], ssem).wait()

    def deg_start(d2ref, dsem):
        pltpu.async_copy(onesb, deg_s.at[d2ref], dsem, add=True)

    def deg_wait(d2ref, dsem):
        pltpu.make_async_copy(onesb, deg_s.at[d2ref], dsem).wait()

    if with_deg:
        set0 = (srcb0, dstb0, d2a, rows0, isem0, gsem0, ssem0, dsem0)
        set1 = (srcb1, dstb1, d2b, rows1, isem1, gsem1, ssem1, dsem1)
    else:
        set0 = (srcb0, dstb0, d2a, rows0, isem0, gsem0, ssem0, None)
        set1 = (srcb1, dstb1, d2b, rows1, isem1, gsem1, ssem1, None)

    def half_iter(b_next, cur, nxt):
        csrc, cdst, cd2, crows, cisem, cgsem, cssem, cdsem = cur
        nsrc, ndst, nd2, nrows, nisem, ngsem, nssem, ndsem = nxt
        idx_wait(nsrc, ndst, nisem)
        if with_deg:
            deg_wait(nd2, ndsem)
        compute_d2(ndst, nd2)
        scatter_wait(nrows, nd2, nssem)
        gather_start(nsrc, nrows, ngsem)
        gather_wait(crows, cgsem)
        scatter_start(crows, cd2, cssem)
        if with_deg:
            deg_start(cd2, cdsem)
        idx_start(b_next, csrc, cdst, cisem)

    idx_start(0, srcb0, dstb0, isem0)
    dfill = jnp.full((16,), dump, jnp.int32)
    for k in range(BK // 16):
        d2b[pl.ds(k * 16, 16)] = dfill
    scatter_start(rows1, d2b, ssem1)
    if with_deg:
        deg_start(d2b, dsem1)
    idx_wait(srcb0, dstb0, isem0)
    compute_d2(dstb0, d2a)
    gather_start(srcb0, rows0, gsem0)
    idx_start(1, srcb1, dstb1, isem1)

    def body(k, carry):
        half_iter(2 * k + 2, set0, set1)
        half_iter(2 * k + 3, set1, set0)
        return carry
    lax.fori_loop(0, (NB - 2) // 2, body, 0)

    idx_wait(srcb1, dstb1, isem1)
    if with_deg:
        deg_wait(d2b, dsem1)
    compute_d2(dstb1, d2b)
    scatter_wait(rows1, d2b, ssem1)
    gather_start(srcb1, rows1, gsem1)
    gather_wait(rows0, gsem0)
    scatter_start(rows0, d2a, ssem0)
    if with_deg:
        deg_start(d2a, dsem0)
    gather_wait(rows1, gsem1)
    scatter_start(rows1, d2b, ssem1)
    if with_deg:
        deg_start(d2b, dsem1)

    scatter_wait(rows0, d2a, ssem0)
    scatter_wait(rows1, d2b, ssem1)
    if with_deg:
        deg_wait(d2a, dsem0)
        deg_wait(d2b, dsem1)

    plsc.subcore_barrier()

    if with_deg:
        def _divide():
            t320 = s * 320

            def grp(g, carry):
                base = t320 + g * 16

                def work(nrows):
                    pltpu.sync_copy(acc_s.at[pl.ds(base, 16)],
                                    rows1.at[pl.ds(0, 16)])
                    pltpu.sync_copy(deg_s.at[pl.ds(base, 16)],
                                    rows0.at[pl.ds(0, 16)])
                    for t in range(16):
                        b16 = 1.0 / jnp.maximum(rows0[t, pl.ds(0, 16)], 1.0)
                        for q in range(F // 16):
                            rows1[t, pl.ds(q * 16, 16)] = (
                                rows1[t, pl.ds(q * 16, 16)] * b16)
                            rows0[t, pl.ds(q * 16, 16)] = b16
                    doff = pl.multiple_of(c * HALF + base, 8)
                    pltpu.sync_copy(rows1.at[pl.ds(0, nrows)],
                                    agg_hbm.at[pl.ds(doff, nrows)])
                    pltpu.sync_copy(rows0.at[pl.ds(0, nrows)],
                                    degb_hbm.at[pl.ds(doff, nrows)])

                @pl.when(base + 16 <= HALF)
                def _full():
                    work(16)

                @pl.when((base < HALF) & (base + 16 > HALF))
                def _partial():
                    work(8)
                return carry
            lax.fori_loop(0, 20, grp, 0)
        _divide()
    else:
        def _divide2():
            t320 = s * 320

            def blk(j, carry):
                base = t320 + j * 40

                @pl.when(base + 40 <= HALF)
                def _():
                    doff = pl.multiple_of(c * HALF + base, 8)
                    pltpu.sync_copy(acc_s.at[pl.ds(base, 40)],
                                    rows1.at[pl.ds(0, 40)])
                    pltpu.sync_copy(degb_hbm.at[pl.ds(doff, 40)],
                                    rows0.at[pl.ds(0, 40)])

                    def rowloop(i, carry2):
                        for q in range(F // 16):
                            rows1[i, pl.ds(q * 16, 16)] = (
                                rows1[i, pl.ds(q * 16, 16)]
                                * rows0[i, pl.ds(q * 16, 16)])
                        return carry2
                    lax.fori_loop(0, 40, rowloop, 0)
                    pltpu.sync_copy(rows1.at[pl.ds(0, 40)],
                                    agg_hbm.at[pl.ds(doff, 40)])
                return carry
            lax.fori_loop(0, 8, blk, 0)
        _divide2()


_SC_MESH = plsc.VectorSubcoreMesh(
    core_axis_name="c", subcore_axis_name="s", num_cores=NC, num_subcores=NS)

_edge_pass_deg = functools.partial(
    pl.kernel, functools.partial(_edge_pass_body, True),
    out_type=[jax.ShapeDtypeStruct((N, F), jnp.float32),
              jax.ShapeDtypeStruct((N, F), jnp.float32)],
    mesh=_SC_MESH,
    scratch_types=[
        pltpu.VMEM_SHARED((AR, F), jnp.float32),
        pltpu.VMEM_SHARED((AR, F), jnp.float32),
        pltpu.VMEM((BK,), jnp.int32),
        pltpu.VMEM((BK,), jnp.int32),
        pltpu.VMEM((BK,), jnp.int32),
        pltpu.VMEM((BK,), jnp.int32),
        pltpu.VMEM((BK,), jnp.int32),
        pltpu.VMEM((BK,), jnp.int32),
        pltpu.VMEM((BK, F), jnp.float32),
        pltpu.VMEM((BK, F), jnp.float32),
        pltpu.VMEM((BK, F), jnp.float32),
        pltpu.SemaphoreType.DMA,
        pltpu.SemaphoreType.DMA,
        pltpu.SemaphoreType.DMA,
        pltpu.SemaphoreType.DMA,
        pltpu.SemaphoreType.DMA,
        pltpu.SemaphoreType.DMA,
        pltpu.SemaphoreType.DMA,
        pltpu.SemaphoreType.DMA,
    ],
)()

_edge_pass = functools.partial(
    pl.kernel, functools.partial(_edge_pass_body, False),
    out_type=jax.ShapeDtypeStruct((N, F), jnp.float32),
    mesh=_SC_MESH,
    scratch_types=[
        pltpu.VMEM_SHARED((AR, F), jnp.float32),
        pltpu.VMEM((BK,), jnp.int32),
        pltpu.VMEM((BK,), jnp.int32),
        pltpu.VMEM((BK,), jnp.int32),
        pltpu.VMEM((BK,), jnp.int32),
        pltpu.VMEM((BK,), jnp.int32),
        pltpu.VMEM((BK,), jnp.int32),
        pltpu.VMEM((BK, F), jnp.float32),
        pltpu.VMEM((BK, F), jnp.float32),
        pltpu.SemaphoreType.DMA,
        pltpu.SemaphoreType.DMA,
        pltpu.SemaphoreType.DMA,
        pltpu.SemaphoreType.DMA,
        pltpu.SemaphoreType.DMA,
        pltpu.SemaphoreType.DMA,
    ],
)()

BR = 400


def _assemble_body(feat_ref, agg1_ref, agg2_ref, out_ref):
    out_ref[:, 0:F] = feat_ref[...]
    out_ref[:, F:2 * F] = agg1_ref[...]
    out_ref[:, 2 * F:3 * F] = agg1_ref[...]
    out_ref[:, 3 * F:4 * F] = agg2_ref[...]


_assemble = pl.pallas_call(
    _assemble_body,
    grid=(N // BR,),
    in_specs=[
        pl.BlockSpec((BR, F), lambda i: (i, 0)),
        pl.BlockSpec((BR, F), lambda i: (i, 0)),
        pl.BlockSpec((BR, F), lambda i: (i, 0)),
    ],
    out_specs=pl.BlockSpec((BR, 4 * F), lambda i: (i, 0)),
    out_shape=jax.ShapeDtypeStruct((N, 4 * F), jnp.float32),
)


def kernel(nodes, edge_index, features):
    src = edge_index[0]
    dst = edge_index[1]
    agg1, degb = _edge_pass_deg(src, dst, features)
    agg2b = _edge_pass(src, dst, agg1, degb)
    return jnp.concatenate([features, agg1, agg1, agg2b], axis=1)


# speedup vs baseline: 6.7827x; 1.0006x over previous
import functools

import jax
import jax.numpy as jnp
from jax import lax
from jax.experimental import pallas as pl
from jax.experimental.pallas import tpu as pltpu
from jax.experimental.pallas import tpu_sc as plsc

N = 10000
E = 320000
F = 128
NC = 2
NS = 16
HALF = N // NC
AR = 5040
EPT = E // NS
BK = 80
NB = EPT // BK


def _edge_pass_body(with_deg, *refs):
    if with_deg:
        (src_hbm, dst_hbm, table_hbm, agg_hbm, degb_hbm,
         acc_s, deg_s, srcb0, srcb1, dstb0, dstb1, d2a, d2b,
         rows0, rows1, onesb, isem0, isem1, gsem0, gsem1,
         ssem0, ssem1, dsem0, dsem1) = refs
    else:
        (src_hbm, dst_hbm, table_hbm, degb_hbm, agg_hbm,
         acc_s, srcb0, srcb1, dstb0, dstb1, d2a, d2b,
         rows0, rows1, isem0, isem1, gsem0, gsem1, ssem0, ssem1) = refs

    c = lax.axis_index("c")
    s = lax.axis_index("s")

    z16 = jnp.zeros((16,), jnp.float32)
    o16 = jnp.ones((16,), jnp.float32)

    def zrow(i, carry):
        for q in range(F // 16):
            rows0[i, pl.ds(q * 16, 16)] = z16
            rows1[i, pl.ds(q * 16, 16)] = z16
        return carry
    lax.fori_loop(0, BK, zrow, 0)

    zoff = s * (AR // NS)
    for j in range(3):
        pltpu.sync_copy(rows0, acc_s.at[pl.ds(zoff + j * BK, BK)])
    pltpu.sync_copy(rows0.at[pl.ds(0, 75)], acc_s.at[pl.ds(zoff + 240, 75)])

    if with_deg:
        for j in range(3):
            pltpu.sync_copy(rows1, deg_s.at[pl.ds(zoff + j * BK, BK)])
        pltpu.sync_copy(rows1.at[pl.ds(0, 75)],
                        deg_s.at[pl.ds(zoff + 240, 75)])

        def orow(i, carry):
            for q in range(F // 16):
                onesb[i, pl.ds(q * 16, 16)] = o16
            return carry
        lax.fori_loop(0, BK, orow, 0)

    plsc.subcore_barrier()

    lo = c * HALF
    dump = HALF + s * 2
    ebase = pl.multiple_of(s * EPT, 8)

    def idx_start(b, srcb, dstb, isem):
        off = pl.multiple_of(ebase + b * BK, 8)
        pltpu.async_copy(src_hbm.at[pl.ds(off, BK)], srcb, isem)
        pltpu.async_copy(dst_hbm.at[pl.ds(off, BK)], dstb, isem)

    def idx_wait(srcb, dstb, isem):
        pltpu.make_async_copy(src_hbm.at[pl.ds(0, BK)], srcb, isem).wait()
        pltpu.make_async_copy(dst_hbm.at[pl.ds(0, BK)], dstb, isem).wait()

    def compute_d2(dstb, d2ref):
        for k in range(BK // 16):
            d = dstb[pl.ds(k * 16, 16)]
            local = d - lo
            inb = (local >= 0) & (local < HALF)
            l2 = jnp.where(inb, local, dump)
            d2ref[pl.ds(k * 16, 16)] = l2

    def gather_start(srcb, rowsref, gsem):
        pltpu.async_copy(table_hbm.at[srcb], rowsref, gsem)

    def gather_wait(rowsref, gsem):
        pltpu.make_async_copy(table_hbm.at[srcb0], rowsref, gsem).wait()

    def scatter_start(rowsref, d2ref, ssem):
        pltpu.async_copy(rowsref, acc_s.at[d2ref], ssem, add=True)

    def scatter_wait(rowsref, d2ref, ssem):
        pltpu.make_async_copy(rowsref, acc_s.at[d2ref], ssem).wait()

    def deg_start(d2ref, dsem):
        pltpu.async_copy(onesb, deg_s.at[d2ref], dsem, add=True)

    def deg_wait(d2ref, dsem):
        pltpu.make_async_copy(onesb, deg_s.at[d2ref], dsem).wait()

    if with_deg:
        set0 = (srcb0, dstb0, d2a, rows0, isem0, gsem0, ssem0, dsem0)
        set1 = (srcb1, dstb1, d2b, rows1, isem1, gsem1, ssem1, dsem1)
    else:
        set0 = (srcb0, dstb0, d2a, rows0, isem0, gsem0, ssem0, None)
        set1 = (srcb1, dstb1, d2b, rows1, isem1, gsem1, ssem1, None)

    def half_iter(b_next, cur, nxt):
        csrc, cdst, cd2, crows, cisem, cgsem, cssem, cdsem = cur
        nsrc, ndst, nd2, nrows, nisem, ngsem, nssem, ndsem = nxt
        idx_wait(nsrc, ndst, nisem)
        if with_deg:
            deg_wait(nd2, ndsem)
        compute_d2(ndst, nd2)
        scatter_wait(nrows, nd2, nssem)
        gather_start(nsrc, nrows, ngsem)
        gather_wait(crows, cgsem)
        scatter_start(crows, cd2, cssem)
        if with_deg:
            deg_start(cd2, cdsem)
        idx_start(b_next, csrc, cdst, cisem)

    idx_start(0, srcb0, dstb0, isem0)
    dfill = jnp.full((16,), dump, jnp.int32)
    for k in range(BK // 16):
        d2b[pl.ds(k * 16, 16)] = dfill
    scatter_start(rows1, d2b, ssem1)
    if with_deg:
        deg_start(d2b, dsem1)
    idx_wait(srcb0, dstb0, isem0)
    compute_d2(dstb0, d2a)
    gather_start(srcb0, rows0, gsem0)
    idx_start(1, srcb1, dstb1, isem1)

    def body(k, carry):
        half_iter(2 * k + 2, set0, set1)
        half_iter(2 * k + 3, set1, set0)
        return carry
    lax.fori_loop(0, (NB - 2) // 2, body, 0)

    idx_wait(srcb1, dstb1, isem1)
    if with_deg:
        deg_wait(d2b, dsem1)
    compute_d2(dstb1, d2b)
    scatter_wait(rows1, d2b, ssem1)
    gather_start(srcb1, rows1, gsem1)
    gather_wait(rows0, gsem0)
    scatter_start(rows0, d2a, ssem0)
    if with_deg:
        deg_start(d2a, dsem0)
    gather_wait(rows1, gsem1)
    scatter_start(rows1, d2b, ssem1)
    if with_deg:
        deg_start(d2b, dsem1)

    scatter_wait(rows0, d2a, ssem0)
    scatter_wait(rows1, d2b, ssem1)
    if with_deg:
        deg_wait(d2a, dsem0)
        deg_wait(d2b, dsem1)

    plsc.subcore_barrier()

    if with_deg:
        def _divide():
            t320 = s * 320

            def grp(g, carry):
                base = t320 + g * 16

                def work(nrows):
                    pltpu.sync_copy(acc_s.at[pl.ds(base, 16)],
                                    rows1.at[pl.ds(0, 16)])
                    pltpu.sync_copy(deg_s.at[pl.ds(base, 16)],
                                    rows0.at[pl.ds(0, 16)])
                    for t in range(16):
                        b16 = 1.0 / jnp.maximum(rows0[t, pl.ds(0, 16)], 1.0)
                        for q in range(F // 16):
                            rows1[t, pl.ds(q * 16, 16)] = (
                                rows1[t, pl.ds(q * 16, 16)] * b16)
                            rows0[t, pl.ds(q * 16, 16)] = b16
                    doff = pl.multiple_of(c * HALF + base, 8)
                    pltpu.sync_copy(rows1.at[pl.ds(0, nrows)],
                                    agg_hbm.at[pl.ds(doff, nrows)])
                    pltpu.sync_copy(rows0.at[pl.ds(0, nrows)],
                                    degb_hbm.at[pl.ds(doff, nrows)])

                @pl.when(base + 16 <= HALF)
                def _full():
                    work(16)

                @pl.when((base < HALF) & (base + 16 > HALF))
                def _partial():
                    work(8)
                return carry
            lax.fori_loop(0, 20, grp, 0)
        _divide()
    else:
        def _divide2():
            t320 = s * 320

            def blk(j, carry):
                base = t320 + j * 40

                @pl.when(base + 40 <= HALF)
                def _():
                    doff = pl.multiple_of(c * HALF + base, 8)
                    pltpu.sync_copy(acc_s.at[pl.ds(base, 40)],
                                    rows1.at[pl.ds(0, 40)])
                    pltpu.sync_copy(degb_hbm.at[pl.ds(doff, 40)],
                                    rows0.at[pl.ds(0, 40)])

                    def rowloop(i, carry2):
                        for q in range(F // 16):
                            rows1[i, pl.ds(q * 16, 16)] = (
                                rows1[i, pl.ds(q * 16, 16)]
                                * rows0[i, pl.ds(q * 16, 16)])
                        return carry2
                    lax.fori_loop(0, 40, rowloop, 0)
                    pltpu.sync_copy(rows1.at[pl.ds(0, 40)],
                                    agg_hbm.at[pl.ds(doff, 40)])
                return carry
            lax.fori_loop(0, 8, blk, 0)
        _divide2()


_SC_MESH = plsc.VectorSubcoreMesh(
    core_axis_name="c", subcore_axis_name="s", num_cores=NC, num_subcores=NS)

_edge_pass_deg = functools.partial(
    pl.kernel, functools.partial(_edge_pass_body, True),
    out_type=[jax.ShapeDtypeStruct((N, F), jnp.float32),
              jax.ShapeDtypeStruct((N, F), jnp.float32)],
    mesh=_SC_MESH,
    scratch_types=[
        pltpu.VMEM_SHARED((AR, F), jnp.float32),
        pltpu.VMEM_SHARED((AR, F), jnp.float32),
        pltpu.VMEM((BK,), jnp.int32),
        pltpu.VMEM((BK,), jnp.int32),
        pltpu.VMEM((BK,), jnp.int32),
        pltpu.VMEM((BK,), jnp.int32),
        pltpu.VMEM((BK,), jnp.int32),
        pltpu.VMEM((BK,), jnp.int32),
        pltpu.VMEM((BK, F), jnp.float32),
        pltpu.VMEM((BK, F), jnp.float32),
        pltpu.VMEM((BK, F), jnp.float32),
        pltpu.SemaphoreType.DMA,
        pltpu.SemaphoreType.DMA,
        pltpu.SemaphoreType.DMA,
        pltpu.SemaphoreType.DMA,
        pltpu.SemaphoreType.DMA,
        pltpu.SemaphoreType.DMA,
        pltpu.SemaphoreType.DMA,
        pltpu.SemaphoreType.DMA,
    ],
)()

_edge_pass = functools.partial(
    pl.kernel, functools.partial(_edge_pass_body, False),
    out_type=jax.ShapeDtypeStruct((N, F), jnp.float32),
    mesh=_SC_MESH,
    scratch_types=[
        pltpu.VMEM_SHARED((AR, F), jnp.float32),
        pltpu.VMEM((BK,), jnp.int32),
        pltpu.VMEM((BK,), jnp.int32),
        pltpu.VMEM((BK,), jnp.int32),
        pltpu.VMEM((BK,), jnp.int32),
        pltpu.VMEM((BK,), jnp.int32),
        pltpu.VMEM((BK,), jnp.int32),
        pltpu.VMEM((BK, F), jnp.float32),
        pltpu.VMEM((BK, F), jnp.float32),
        pltpu.SemaphoreType.DMA,
        pltpu.SemaphoreType.DMA,
        pltpu.SemaphoreType.DMA,
        pltpu.SemaphoreType.DMA,
        pltpu.SemaphoreType.DMA,
        pltpu.SemaphoreType.DMA,
    ],
)()

def kernel(nodes, edge_index, features):
    src = edge_index[0]
    dst = edge_index[1]
    agg1, degb = _edge_pass_deg(src, dst, features)
    agg2b = _edge_pass(src, dst, agg1, degb)
    return jnp.concatenate([features, agg1, agg1, agg2b], axis=1)
